# baseline (speedup 1.0000x reference)
import functools

import jax
import jax.numpy as jnp
from jax import lax
from jax.experimental import pallas as pl
from jax.experimental.pallas import tpu as pltpu
from jax.experimental.pallas import tpu_sc as plsc

D = 1024
NC = 2
NS = 16
NW = NC * NS
B = 4 * 8192
S = 8192
B_PER_W = B // NW
W_PER_BATCH = S // B_PER_W
CHUNK = 32
N_CHUNKS = B_PER_W // CHUNK
NB = 3
LA = 1

_mesh = plsc.VectorSubcoreMesh(core_axis_name="c", subcore_axis_name="s")


@functools.partial(
    pl.kernel,
    mesh=_mesh,
    out_type=jax.ShapeDtypeStruct((B, D), jnp.float32),
    scratch_types=(
        [pltpu.VMEM((1, B_PER_W), jnp.int32)]
        + [pltpu.VMEM((CHUNK, D), jnp.float32) for _ in range(NB)]
        + [pltpu.SemaphoreType.DMA for _ in range(2 * NB)]
    ),
)
def _gather_kernel(t_hbm, pe_hbm, out_hbm, idx_v, *bufs):
    rows = bufs[:NB]
    gsem = bufs[NB:2 * NB]
    ssem = bufs[2 * NB:]

    wid = lax.axis_index("s") * NC + lax.axis_index("c")
    base = wid * B_PER_W
    bi = wid // W_PER_BATCH
    s0 = (wid % W_PER_BATCH) * B_PER_W
    pltpu.sync_copy(t_hbm.at[pl.ds(bi, 1), pl.ds(s0, B_PER_W)], idx_v)

    def idx_slice(j):
        return idx_v.at[0, pl.ds(j * CHUNK, CHUNK)]

    def g_start(j, b):
        pltpu.async_copy(pe_hbm.at[idx_slice(j)], rows[b], gsem[b])

    def g_wait(j, b):
        pltpu.make_async_copy(pe_hbm.at[idx_slice(j)], rows[b], gsem[b]).wait()

    def out_slice(j):
        return out_hbm.at[pl.ds(base + j * CHUNK, CHUNK)]

    def s_start(j, b):
        pltpu.async_copy(rows[b], out_slice(j), ssem[b])

    def s_wait(j, b):
        pltpu.make_async_copy(rows[b], out_slice(j), ssem[b]).wait()

    def step(j, jb, full_wait=True, start_next=True):
        g_wait(j, jb)
        s_start(j, jb)
        if NB > 2 * LA:
            if start_next:
                g_start(j + LA, (jb + LA) % NB)
            if full_wait:
                s_wait(j - LA, (jb - LA) % NB)
        else:
            if full_wait:
                s_wait(j - LA, (jb - LA) % NB)
            if start_next:
                g_start(j + LA, (jb + LA) % NB)

    for j in range(LA):
        g_start(j, j % NB)
    for j in range(LA):
        step(j, j % NB, full_wait=False)

    K = (N_CHUNKS - 2 * LA) // NB

    def body(k, carry):
        jbase = LA + k * NB
        for b in range(NB):
            step(jbase + b, (LA + b) % NB)
        return carry

    lax.fori_loop(0, K, body, 0)

    for j in range(LA + K * NB, N_CHUNKS):
        step(j, j % NB, start_next=(j + LA < N_CHUNKS))
    for j in range(N_CHUNKS - LA, N_CHUNKS):
        s_wait(j, j % NB)


def kernel(t, pe):
    out = _gather_kernel(t.astype(jnp.int32), pe)
    return out.reshape(t.shape[0], t.shape[1], D)


# speedup vs baseline: 1.0394x; 1.0394x over previous
import functools

import jax
import jax.numpy as jnp
from jax import lax
from jax.experimental import pallas as pl
from jax.experimental.pallas import tpu as pltpu
from jax.experimental.pallas import tpu_sc as plsc

D = 1024
NC = 2
NS = 16
NW = NC * NS
B = 4 * 8192
S = 8192
B_PER_W = B // NW
W_PER_BATCH = S // B_PER_W
CHUNK = 8
N_CHUNKS = B_PER_W // CHUNK
NB = 12
LA = 4

_mesh = plsc.VectorSubcoreMesh(core_axis_name="c", subcore_axis_name="s")


@functools.partial(
    pl.kernel,
    mesh=_mesh,
    out_type=jax.ShapeDtypeStruct((B, D), jnp.float32),
    scratch_types=(
        [pltpu.VMEM((1, B_PER_W), jnp.int32)]
        + [pltpu.VMEM((CHUNK, D), jnp.float32) for _ in range(NB)]
        + [pltpu.SemaphoreType.DMA for _ in range(2 * NB)]
    ),
)
def _gather_kernel(t_hbm, pe_hbm, out_hbm, idx_v, *bufs):
    rows = bufs[:NB]
    gsem = bufs[NB:2 * NB]
    ssem = bufs[2 * NB:]

    wid = lax.axis_index("s") * NC + lax.axis_index("c")
    base = wid * B_PER_W
    bi = wid // W_PER_BATCH
    s0 = (wid % W_PER_BATCH) * B_PER_W
    pltpu.sync_copy(t_hbm.at[pl.ds(bi, 1), pl.ds(s0, B_PER_W)], idx_v)

    def idx_slice(j):
        return idx_v.at[0, pl.ds(j * CHUNK, CHUNK)]

    def g_start(j, b):
        pltpu.async_copy(pe_hbm.at[idx_slice(j)], rows[b], gsem[b])

    def g_wait(j, b):
        pltpu.make_async_copy(pe_hbm.at[idx_slice(j)], rows[b], gsem[b]).wait()

    def out_slice(j):
        return out_hbm.at[pl.ds(base + j * CHUNK, CHUNK)]

    def s_start(j, b):
        pltpu.async_copy(rows[b], out_slice(j), ssem[b])

    def s_wait(j, b):
        pltpu.make_async_copy(rows[b], out_slice(j), ssem[b]).wait()

    def step(j, jb, full_wait=True, start_next=True):
        g_wait(j, jb)
        s_start(j, jb)
        if NB > 2 * LA:
            if start_next:
                g_start(j + LA, (jb + LA) % NB)
            if full_wait:
                s_wait(j - LA, (jb - LA) % NB)
        else:
            if full_wait:
                s_wait(j - LA, (jb - LA) % NB)
            if start_next:
                g_start(j + LA, (jb + LA) % NB)

    for j in range(LA):
        g_start(j, j % NB)
    for j in range(LA):
        step(j, j % NB, full_wait=False)

    K = (N_CHUNKS - 2 * LA) // NB

    def body(k, carry):
        jbase = LA + k * NB
        for b in range(NB):
            step(jbase + b, (LA + b) % NB)
        return carry

    lax.fori_loop(0, K, body, 0)

    for j in range(LA + K * NB, N_CHUNKS):
        step(j, j % NB, start_next=(j + LA < N_CHUNKS))
    for j in range(N_CHUNKS - LA, N_CHUNKS):
        s_wait(j, j % NB)


def kernel(t, pe):
    out = _gather_kernel(t.astype(jnp.int32), pe)
    return out.reshape(t.shape[0], t.shape[1], D)


# speedup vs baseline: 1.0427x; 1.0032x over previous
import functools

import jax
import jax.numpy as jnp
from jax import lax
from jax.experimental import pallas as pl
from jax.experimental.pallas import tpu as pltpu
from jax.experimental.pallas import tpu_sc as plsc

D = 1024
NC = 2
NS = 16
NW = NC * NS
B = 4 * 8192
S = 8192
B_PER_W = B // NW
W_PER_BATCH = S // B_PER_W
CHUNK = 16
N_CHUNKS = B_PER_W // CHUNK
NB = 6
LA = 3

_mesh = plsc.VectorSubcoreMesh(core_axis_name="c", subcore_axis_name="s")


@functools.partial(
    pl.kernel,
    mesh=_mesh,
    out_type=jax.ShapeDtypeStruct((B, D), jnp.float32),
    scratch_types=(
        [pltpu.VMEM((1, B_PER_W), jnp.int32)]
        + [pltpu.VMEM((CHUNK, D), jnp.float32) for _ in range(NB)]
        + [pltpu.SemaphoreType.DMA for _ in range(2 * NB)]
    ),
)
def _gather_kernel(t_hbm, pe_hbm, out_hbm, idx_v, *bufs):
    rows = bufs[:NB]
    gsem = bufs[NB:2 * NB]
    ssem = bufs[2 * NB:]

    wid = lax.axis_index("s") * NC + lax.axis_index("c")
    base = wid * B_PER_W
    bi = wid // W_PER_BATCH
    s0 = (wid % W_PER_BATCH) * B_PER_W
    pltpu.sync_copy(t_hbm.at[pl.ds(bi, 1), pl.ds(s0, B_PER_W)], idx_v)

    def idx_slice(j):
        return idx_v.at[0, pl.ds(j * CHUNK, CHUNK)]

    def g_start(j, b):
        pltpu.async_copy(pe_hbm.at[idx_slice(j)], rows[b], gsem[b])

    def g_wait(j, b):
        pltpu.make_async_copy(pe_hbm.at[idx_slice(j)], rows[b], gsem[b]).wait()

    def out_slice(j):
        return out_hbm.at[pl.ds(base + j * CHUNK, CHUNK)]

    def s_start(j, b):
        pltpu.async_copy(rows[b], out_slice(j), ssem[b])

    def s_wait(j, b):
        pltpu.make_async_copy(rows[b], out_slice(j), ssem[b]).wait()

    def step(j, jb, full_wait=True, start_next=True):
        g_wait(j, jb)
        s_start(j, jb)
        if NB > 2 * LA:
            if start_next:
                g_start(j + LA, (jb + LA) % NB)
            if full_wait:
                s_wait(j - LA, (jb - LA) % NB)
        else:
            if full_wait:
                s_wait(j - LA, (jb - LA) % NB)
            if start_next:
                g_start(j + LA, (jb + LA) % NB)

    for j in range(LA):
        g_start(j, j % NB)
    for j in range(LA):
        step(j, j % NB, full_wait=False)

    K = (N_CHUNKS - 2 * LA) // NB

    def body(k, carry):
        jbase = LA + k * NB
        for b in range(NB):
            step(jbase + b, (LA + b) % NB)
        return carry

    lax.fori_loop(0, K, body, 0)

    for j in range(LA + K * NB, N_CHUNKS):
        step(j, j % NB, start_next=(j + LA < N_CHUNKS))
    for j in range(N_CHUNKS - LA, N_CHUNKS):
        s_wait(j, j % NB)


def kernel(t, pe):
    out = _gather_kernel(t.astype(jnp.int32), pe)
    return out.reshape(t.shape[0], t.shape[1], D)
